# baseline (speedup 1.0000x reference)
import functools

import jax
import jax.numpy as jnp
from jax import lax
from jax.experimental import pallas as pl
from jax.experimental.pallas import tpu as pltpu
from jax.experimental.pallas import tpu_sc as plsc

_LANES = 128
_TOK_BLK = 2048


@functools.lru_cache(maxsize=None)
def _flags_gather(b, t, nc, ns):
    n = b * t
    nw = nc * ns
    n_w = n // nw
    rows_w = n_w // _LANES
    w_per_b = t // n_w
    mesh = plsc.VectorSubcoreMesh(core_axis_name="c", subcore_axis_name="s")

    def body(ids_hbm, table_hbm, out_hbm, dirty_hbm, idx_v, fl_v, cnt_v, sem):
        wid = lax.axis_index("s") * nc + lax.axis_index("c")
        bi = wid // w_per_b
        toff = (wid % w_per_b) * n_w
        pltpu.sync_copy(ids_hbm.at[bi, pl.ds(toff, n_w)], idx_v)
        copies = [
            pltpu.async_copy(
                table_hbm.at[idx_v.at[pl.ds(j * _LANES, _LANES)]],
                fl_v.at[pl.ds(j * _LANES, _LANES)], sem)
            for j in range(rows_w)
        ]
        for cp in copies:
            cp.wait()
        acc = jnp.zeros((16,), jnp.int32)
        for k in range(n_w // 16):
            acc = acc | fl_v[pl.ds(k * 16, 16)]
        cnt_v[...] = acc
        pltpu.sync_copy(fl_v, out_hbm.at[0, pl.ds(wid * n_w, n_w)])
        pltpu.sync_copy(cnt_v, dirty_hbm.at[pl.ds(wid * 16, 16)])

    return pl.kernel(
        body,
        out_type=(
            jax.ShapeDtypeStruct((1, n), jnp.int32),
            jax.ShapeDtypeStruct((nw * 16,), jnp.int32),
        ),
        mesh=mesh,
        scratch_types=[
            pltpu.VMEM((n_w,), jnp.int32),
            pltpu.VMEM((n_w,), jnp.int32),
            pltpu.VMEM((16,), jnp.int32),
            pltpu.SemaphoreType.DMA,
        ],
    )


def _make_add_body(scal_per_blk):
    def _add_body(dirty_ref, fl_ref, se_ref, emb_ref, out_ref):
        i = pl.program_id(0)
        dirty = jnp.int32(0)
        for k in range(scal_per_blk * 16):
            dirty = dirty | dirty_ref[i * scal_per_blk * 16 + k]

        @pl.when(dirty > 0)
        def _dense():
            f = fl_ref[...]
            k8 = lax.broadcasted_iota(jnp.int32, (8, 1), 0)
            oh = (f == k8).astype(jnp.float32)
            sp = lax.dot_general(
                oh, se_ref[...],
                dimension_numbers=(((0,), (0,)), ((), ())),
                preferred_element_type=jnp.float32)
            out_ref[...] = emb_ref[...] + sp

        @pl.when(dirty == 0)
        def _copy():
            out_ref[...] = emb_ref[...]

    return _add_body


def kernel(token_embeds, token_ids, special_flags, special_emb):
    b, t, d = token_embeds.shape
    n = b * t
    info = plsc.get_sparse_core_info()
    nc, ns = info.num_cores, info.num_subcores
    nw = nc * ns

    flags_row, dirty = _flags_gather(b, t, nc, ns)(
        token_ids.astype(jnp.int32), special_flags.astype(jnp.int32))

    se8 = jnp.zeros((8, d), jnp.float32).at[:3].set(special_emb)
    emb2d = token_embeds.reshape(n, d)
    scal_per_blk = _TOK_BLK // (n // nw)
    grid_spec = pltpu.PrefetchScalarGridSpec(
        num_scalar_prefetch=1,
        grid=(n // _TOK_BLK,),
        in_specs=[
            pl.BlockSpec((1, _TOK_BLK), lambda i, s: (0, i)),
            pl.BlockSpec((8, d), lambda i, s: (0, 0)),
            pl.BlockSpec((_TOK_BLK, d), lambda i, s: (i, 0)),
        ],
        out_specs=pl.BlockSpec((_TOK_BLK, d), lambda i, s: (i, 0)),
    )
    out2d = pl.pallas_call(
        _make_add_body(scal_per_blk),
        grid_spec=grid_spec,
        out_shape=jax.ShapeDtypeStruct((n, d), jnp.float32),
        compiler_params=pltpu.CompilerParams(
            dimension_semantics=("arbitrary",)),
    )(dirty, flags_row, se8, emb2d)
    return out2d.reshape(b, t, d)


# speedup vs baseline: 1.0148x; 1.0148x over previous
import functools

import jax
import jax.numpy as jnp
from jax import lax
from jax.experimental import pallas as pl
from jax.experimental.pallas import tpu as pltpu
from jax.experimental.pallas import tpu_sc as plsc

_LANES = 128
_TOK_BLK = 2048


@functools.lru_cache(maxsize=None)
def _flags_gather(b, t, nc, ns):
    n = b * t
    nw = nc * ns
    n_w = n // nw
    rows_w = n_w // _LANES
    w_per_b = t // n_w
    w_per_blk = _TOK_BLK // n_w
    mesh = plsc.VectorSubcoreMesh(core_axis_name="c", subcore_axis_name="s")

    def body(ids_hbm, table_hbm, out_hbm, dirty_hbm, idx_v, fl_v, cnt_v, sem):
        wid = lax.axis_index("s") * nc + lax.axis_index("c")
        bi = wid // w_per_b
        toff = (wid % w_per_b) * n_w
        pltpu.sync_copy(ids_hbm.at[bi, pl.ds(toff, n_w)], idx_v)
        copies = [
            pltpu.async_copy(
                table_hbm.at[idx_v.at[pl.ds(j * _LANES, _LANES)]],
                fl_v.at[pl.ds(j * _LANES, _LANES)], sem)
            for j in range(rows_w)
        ]
        for cp in copies:
            cp.wait()
        acc = jnp.zeros((16,), jnp.int32)
        for k in range(n_w // 16):
            acc = acc | fl_v[pl.ds(k * 16, 16)]
        cnt_v[...] = acc
        pltpu.sync_copy(
            fl_v,
            out_hbm.at[wid // w_per_blk,
                       pl.ds((wid % w_per_blk) * n_w, n_w)])
        pltpu.sync_copy(cnt_v, dirty_hbm.at[pl.ds(wid * 16, 16)])

    return pl.kernel(
        body,
        out_type=(
            jax.ShapeDtypeStruct((n // _TOK_BLK, _TOK_BLK), jnp.int32),
            jax.ShapeDtypeStruct((nw * 16,), jnp.int32),
        ),
        mesh=mesh,
        scratch_types=[
            pltpu.VMEM((n_w,), jnp.int32),
            pltpu.VMEM((n_w,), jnp.int32),
            pltpu.VMEM((16,), jnp.int32),
            pltpu.SemaphoreType.DMA,
        ],
    )


def _make_add_body(scal_per_blk):
    def _add_body(dirty_ref, fl_ref, se_ref, emb_ref, out_ref):
        i = pl.program_id(0)
        dirty = jnp.int32(0)
        for k in range(scal_per_blk * 16):
            dirty = dirty | dirty_ref[i * scal_per_blk * 16 + k]

        @pl.when(dirty > 0)
        def _dense():
            f = fl_ref[pl.ds(i, 1), :]
            k8 = lax.broadcasted_iota(jnp.int32, (8, 1), 0)
            oh = (f == k8).astype(jnp.float32)
            sp = lax.dot_general(
                oh, se_ref[...],
                dimension_numbers=(((0,), (0,)), ((), ())),
                preferred_element_type=jnp.float32)
            out_ref[...] = emb_ref[...] + sp

        @pl.when(dirty == 0)
        def _copy():
            out_ref[...] = emb_ref[...]

    return _add_body


def kernel(token_embeds, token_ids, special_flags, special_emb):
    b, t, d = token_embeds.shape
    n = b * t
    info = plsc.get_sparse_core_info()
    nc, ns = info.num_cores, info.num_subcores
    nw = nc * ns
    nblk = n // _TOK_BLK

    flags2d, dirty = _flags_gather(b, t, nc, ns)(
        token_ids.astype(jnp.int32), special_flags.astype(jnp.int32))

    se8 = jnp.zeros((8, d), jnp.float32).at[:3].set(special_emb)
    emb2d = token_embeds.reshape(n, d)
    scal_per_blk = _TOK_BLK // (n // nw)
    grid_spec = pltpu.PrefetchScalarGridSpec(
        num_scalar_prefetch=1,
        grid=(nblk,),
        in_specs=[
            pl.BlockSpec((nblk, _TOK_BLK), lambda i, s: (0, 0)),
            pl.BlockSpec((8, d), lambda i, s: (0, 0)),
            pl.BlockSpec((_TOK_BLK, d), lambda i, s: (i, 0)),
        ],
        out_specs=pl.BlockSpec((_TOK_BLK, d), lambda i, s: (i, 0)),
    )
    out2d = pl.pallas_call(
        _make_add_body(scal_per_blk),
        grid_spec=grid_spec,
        out_shape=jax.ShapeDtypeStruct((n, d), jnp.float32),
        compiler_params=pltpu.CompilerParams(
            dimension_semantics=("arbitrary",)),
    )(dirty, flags2d, se8, emb2d)
    return out2d.reshape(b, t, d)
